# Initial kernel scaffold; baseline (speedup 1.0000x reference)
import jax
import jax.numpy as jnp
from jax.experimental import pallas as pl


def kernel(x, edge_index, W1, b1, W2, b2):
    raise NotImplementedError("write your pallas kernel here")


# speedup vs baseline: 16.2096x; 16.2096x over previous
import functools

import jax
import jax.numpy as jnp
from jax import lax
from jax.experimental import pallas as pl
from jax.experimental.pallas import tpu as pltpu
from jax.experimental.pallas import tpu_sc as plsc

N = 10000
NP = 10240
E = 320000
D = 128
NC, NS = 2, 16
NW = NC * NS
EPW = E // NW
KB = 80
NJ = EPW // KB
CN = 25
NCH = NJ // CN
RPT = NP // NS
BM = 512

_mesh = plsc.VectorSubcoreMesh(
    core_axis_name="c", subcore_axis_name="s", num_cores=NC, num_subcores=NS)


@functools.partial(
    pl.kernel,
    out_type=jax.ShapeDtypeStruct((NC * NP, D), jnp.float32),
    mesh=_mesh,
    scratch_types=[
        pltpu.VMEM((CN, KB), jnp.int32),
        pltpu.VMEM((KB, D), jnp.float32),
        pltpu.VMEM((KB, D), jnp.float32),
        pltpu.VMEM_SHARED((NP, D), jnp.float32),
    ],
)
def _deg_kernel(dst_hbm, ones_hbm, zD_hbm, out_hbm, dst_v, ones_v, z_v, acc_sh):
    cid = lax.axis_index("c")
    sid = lax.axis_index("s")
    wid = cid * NS + sid
    pltpu.sync_copy(ones_hbm, ones_v)
    pltpu.sync_copy(zD_hbm, z_v)
    rows0 = sid * RPT
    for i in range(RPT // KB):
        pltpu.sync_copy(z_v, acc_sh.at[pl.ds(rows0 + i * KB, KB)])
    plsc.subcore_barrier()

    def chunk_body(c, carry):
        pltpu.sync_copy(dst_hbm.at[wid * NCH + c], dst_v)

        def body(j, cc):
            pltpu.sync_copy(ones_v, acc_sh.at[dst_v.at[j]], add=True)
            return cc

        lax.fori_loop(0, CN, body, 0)
        return carry

    lax.fori_loop(0, NCH, chunk_body, 0)
    plsc.subcore_barrier()
    for i in range(RPT // KB):
        pltpu.sync_copy(acc_sh.at[pl.ds(rows0 + i * KB, KB)], z_v)
        pltpu.sync_copy(
            z_v, out_hbm.at[pl.ds(cid * NP + rows0 + i * KB, KB)])


@functools.partial(
    pl.kernel,
    out_type=jax.ShapeDtypeStruct((NC * NP, D), jnp.float32),
    mesh=_mesh,
    scratch_types=[
        pltpu.VMEM((CN, KB), jnp.int32),
        pltpu.VMEM((CN, KB), jnp.int32),
        pltpu.VMEM((KB, D), jnp.float32),
        pltpu.VMEM((KB, D), jnp.float32),
        pltpu.VMEM_SHARED((NP, D), jnp.float32),
        pltpu.SemaphoreType.DMA,
        pltpu.SemaphoreType.DMA,
    ],
)
def _agg_kernel(g_hbm, src_hbm, dst_hbm, zD_hbm, out_hbm,
                src_v, dst_v, rows_a, rows_b, acc_sh, sem_a, sem_b):
    cid = lax.axis_index("c")
    sid = lax.axis_index("s")
    wid = cid * NS + sid
    rows0 = sid * RPT
    pltpu.sync_copy(zD_hbm, rows_a)
    for i in range(RPT // KB):
        pltpu.sync_copy(rows_a, acc_sh.at[pl.ds(rows0 + i * KB, KB)])
    plsc.subcore_barrier()

    def chunk_body(c, carry):
        pltpu.sync_copy(src_hbm.at[wid * NCH + c], src_v)
        pltpu.sync_copy(dst_hbm.at[wid * NCH + c], dst_v)

        def step(j, cc):
            pltpu.async_copy(g_hbm.at[src_v.at[j]], rows_a, sem_a).wait()
            pltpu.sync_copy(rows_a, acc_sh.at[dst_v.at[j]], add=True)
            return cc

        lax.fori_loop(0, CN, step, 0)
        return carry

    lax.fori_loop(0, NCH, chunk_body, 0)

    plsc.subcore_barrier()
    for i in range(RPT // KB):
        pltpu.sync_copy(acc_sh.at[pl.ds(rows0 + i * KB, KB)], rows_a)
        pltpu.sync_copy(
            rows_a, out_hbm.at[pl.ds(cid * NP + rows0 + i * KB, KB)])


def _dinv_block(dega_ref, degb_ref, i):
    deg = dega_ref[:, :1] + degb_ref[:, :1]
    rowid = i * BM + lax.broadcasted_iota(jnp.int32, (BM, 1), 0)
    deg = deg + jnp.where(rowid < N, 1.0, 0.0)
    return jnp.where(deg > 0, lax.rsqrt(deg), 0.0)


def _tc1_body(x_ref, w_ref, dega_ref, degb_ref, g_ref):
    dinv = _dinv_block(dega_ref, degb_ref, pl.program_id(0))
    h = jnp.dot(x_ref[...], w_ref[...], preferred_element_type=jnp.float32)
    g_ref[...] = h * dinv


def _tc2_body(acca_ref, accb_ref, g1_ref, dega_ref, degb_ref, w_ref, b_ref,
              g2_ref):
    dinv = _dinv_block(dega_ref, degb_ref, pl.program_id(0))
    z = dinv * (acca_ref[...] + accb_ref[...] + g1_ref[...]) + b_ref[...]
    z = jnp.maximum(z, 0.0)
    h2 = jnp.dot(z, w_ref[...], preferred_element_type=jnp.float32)
    g2_ref[...] = h2 * dinv


def _tc3_body(acca_ref, accb_ref, g2_ref, dega_ref, degb_ref, b_ref, out_ref):
    dinv = _dinv_block(dega_ref, degb_ref, pl.program_id(0))
    out_ref[...] = dinv * (acca_ref[...] + accb_ref[...] + g2_ref[...]) \
        + b_ref[...]


_row_spec = pl.BlockSpec((BM, D), lambda i: (i, 0))
_w_spec = pl.BlockSpec((D, D), lambda i: (0, 0))
_b_spec = pl.BlockSpec((1, D), lambda i: (0, 0))
_GRID = (NP // BM,)
_OUT = jax.ShapeDtypeStruct((NP, D), jnp.float32)

_tc1 = pl.pallas_call(
    _tc1_body, grid=_GRID, out_shape=_OUT,
    in_specs=[_row_spec, _w_spec, _row_spec, _row_spec],
    out_specs=_row_spec)
_tc2 = pl.pallas_call(
    _tc2_body, grid=_GRID, out_shape=_OUT,
    in_specs=[_row_spec, _row_spec, _row_spec, _row_spec, _row_spec,
              _w_spec, _b_spec],
    out_specs=_row_spec)
_tc3 = pl.pallas_call(
    _tc3_body, grid=_GRID, out_shape=_OUT,
    in_specs=[_row_spec, _row_spec, _row_spec, _row_spec, _row_spec, _b_spec],
    out_specs=_row_spec)


def kernel(x, edge_index, W1, b1, W2, b2):
    ei = edge_index.astype(jnp.int32)
    src = ei[0].reshape(NW * NCH, CN, KB)
    dst = ei[1].reshape(NW * NCH, CN, KB)
    x_p = jnp.pad(x, ((0, NP - N), (0, 0)))
    onesD = jnp.zeros((KB, D), jnp.float32).at[:, 0].set(1.0)
    zD = jnp.zeros((KB, D), jnp.float32)
    b1r = b1.reshape(1, D)
    b2r = b2.reshape(1, D)

    degD = _deg_kernel(dst, onesD, zD)
    dega, degb = degD[:NP], degD[NP:]

    g1 = _tc1(x_p, W1, dega, degb)
    acc1 = _agg_kernel(g1, src, dst, zD)
    g2 = _tc2(acc1[:NP], acc1[NP:], g1, dega, degb, W2, b1r)
    acc2 = _agg_kernel(g2, src, dst, zD)
    out = _tc3(acc2[:NP], acc2[NP:], g2, dega, degb, b2r)
    return out[:N]


# speedup vs baseline: 22.1998x; 1.3696x over previous
import functools

import jax
import jax.numpy as jnp
from jax import lax
from jax.experimental import pallas as pl
from jax.experimental.pallas import tpu as pltpu
from jax.experimental.pallas import tpu_sc as plsc

N = 10000
NP = 10240
E = 320000
D = 128
NC, NS = 2, 16
NW = NC * NS
EPW = E // NW
KB = 80
NJ = EPW // KB
CN = 25
NCH = NJ // CN
RPT = NP // NS
BM = 512

_mesh = plsc.VectorSubcoreMesh(
    core_axis_name="c", subcore_axis_name="s", num_cores=NC, num_subcores=NS)


@functools.partial(
    pl.kernel,
    out_type=jax.ShapeDtypeStruct((NC * NP, D), jnp.float32),
    mesh=_mesh,
    scratch_types=[
        pltpu.VMEM((CN, KB), jnp.int32),
        pltpu.VMEM((KB, D), jnp.float32),
        pltpu.VMEM((KB, D), jnp.float32),
        pltpu.VMEM_SHARED((NP, D), jnp.float32),
    ],
)
def _deg_kernel(dst_hbm, ones_hbm, zD_hbm, out_hbm, dst_v, ones_v, z_v, acc_sh):
    cid = lax.axis_index("c")
    sid = lax.axis_index("s")
    wid = cid * NS + sid
    pltpu.sync_copy(ones_hbm, ones_v)
    pltpu.sync_copy(zD_hbm, z_v)
    rows0 = sid * RPT
    for i in range(RPT // KB):
        pltpu.sync_copy(z_v, acc_sh.at[pl.ds(rows0 + i * KB, KB)])
    plsc.subcore_barrier()

    def chunk_body(c, carry):
        pltpu.sync_copy(dst_hbm.at[wid * NCH + c], dst_v)

        def body(j, cc):
            pltpu.sync_copy(ones_v, acc_sh.at[dst_v.at[j]], add=True)
            return cc

        lax.fori_loop(0, CN, body, 0)
        return carry

    lax.fori_loop(0, NCH, chunk_body, 0)
    plsc.subcore_barrier()
    for i in range(RPT // KB):
        pltpu.sync_copy(acc_sh.at[pl.ds(rows0 + i * KB, KB)], z_v)
        pltpu.sync_copy(
            z_v, out_hbm.at[pl.ds(cid * NP + rows0 + i * KB, KB)])


@functools.partial(
    pl.kernel,
    out_type=jax.ShapeDtypeStruct((NC * NP, D), jnp.float32),
    mesh=_mesh,
    scratch_types=[
        pltpu.VMEM((CN, KB), jnp.int32),
        pltpu.VMEM((CN, KB), jnp.int32),
        pltpu.VMEM((KB, D), jnp.float32),
        pltpu.VMEM((KB, D), jnp.float32),
        pltpu.VMEM_SHARED((NP, D), jnp.float32),
        pltpu.SemaphoreType.DMA,
        pltpu.SemaphoreType.DMA,
    ],
)
def _agg_kernel(g_hbm, src_hbm, dst_hbm, zD_hbm, out_hbm,
                src_v, dst_v, rows_a, rows_b, acc_sh, sem_a, sem_b):
    cid = lax.axis_index("c")
    sid = lax.axis_index("s")
    wid = cid * NS + sid
    rows0 = sid * RPT
    pltpu.sync_copy(zD_hbm, rows_a)
    for i in range(RPT // KB):
        pltpu.sync_copy(rows_a, acc_sh.at[pl.ds(rows0 + i * KB, KB)])
    plsc.subcore_barrier()

    def chunk_body(c, carry):
        pltpu.sync_copy(src_hbm.at[wid * NCH + c], src_v)
        pltpu.sync_copy(dst_hbm.at[wid * NCH + c], dst_v)
        pltpu.async_copy(g_hbm.at[src_v.at[0]], rows_a, sem_a)

        def pair(j2, cc):
            ja = 2 * j2
            jb = ja + 1
            pltpu.async_copy(g_hbm.at[src_v.at[jb]], rows_b, sem_b)
            pltpu.make_async_copy(g_hbm.at[src_v.at[ja]], rows_a, sem_a).wait()
            pltpu.sync_copy(rows_a, acc_sh.at[dst_v.at[ja]], add=True)
            pltpu.async_copy(g_hbm.at[src_v.at[ja + 2]], rows_a, sem_a)
            pltpu.make_async_copy(g_hbm.at[src_v.at[jb]], rows_b, sem_b).wait()
            pltpu.sync_copy(rows_b, acc_sh.at[dst_v.at[jb]], add=True)
            return cc

        lax.fori_loop(0, CN // 2, pair, 0)
        pltpu.make_async_copy(g_hbm.at[src_v.at[CN - 1]], rows_a, sem_a).wait()
        pltpu.sync_copy(rows_a, acc_sh.at[dst_v.at[CN - 1]], add=True)
        return carry

    lax.fori_loop(0, NCH, chunk_body, 0)

    plsc.subcore_barrier()
    for i in range(RPT // KB):
        pltpu.sync_copy(acc_sh.at[pl.ds(rows0 + i * KB, KB)], rows_a)
        pltpu.sync_copy(
            rows_a, out_hbm.at[pl.ds(cid * NP + rows0 + i * KB, KB)])


def _dinv_block(dega_ref, degb_ref, i):
    deg = dega_ref[:, :1] + degb_ref[:, :1]
    rowid = i * BM + lax.broadcasted_iota(jnp.int32, (BM, 1), 0)
    deg = deg + jnp.where(rowid < N, 1.0, 0.0)
    return jnp.where(deg > 0, lax.rsqrt(deg), 0.0)


def _tc1_body(x_ref, w_ref, dega_ref, degb_ref, g_ref):
    dinv = _dinv_block(dega_ref, degb_ref, pl.program_id(0))
    h = jnp.dot(x_ref[...], w_ref[...], preferred_element_type=jnp.float32)
    g_ref[...] = h * dinv


def _tc2_body(acca_ref, accb_ref, g1_ref, dega_ref, degb_ref, w_ref, b_ref,
              g2_ref):
    dinv = _dinv_block(dega_ref, degb_ref, pl.program_id(0))
    z = dinv * (acca_ref[...] + accb_ref[...] + g1_ref[...]) + b_ref[...]
    z = jnp.maximum(z, 0.0)
    h2 = jnp.dot(z, w_ref[...], preferred_element_type=jnp.float32)
    g2_ref[...] = h2 * dinv


def _tc3_body(acca_ref, accb_ref, g2_ref, dega_ref, degb_ref, b_ref, out_ref):
    dinv = _dinv_block(dega_ref, degb_ref, pl.program_id(0))
    out_ref[...] = dinv * (acca_ref[...] + accb_ref[...] + g2_ref[...]) \
        + b_ref[...]


_row_spec = pl.BlockSpec((BM, D), lambda i: (i, 0))
_w_spec = pl.BlockSpec((D, D), lambda i: (0, 0))
_b_spec = pl.BlockSpec((1, D), lambda i: (0, 0))
_GRID = (NP // BM,)
_OUT = jax.ShapeDtypeStruct((NP, D), jnp.float32)

_tc1 = pl.pallas_call(
    _tc1_body, grid=_GRID, out_shape=_OUT,
    in_specs=[_row_spec, _w_spec, _row_spec, _row_spec],
    out_specs=_row_spec)
_tc2 = pl.pallas_call(
    _tc2_body, grid=_GRID, out_shape=_OUT,
    in_specs=[_row_spec, _row_spec, _row_spec, _row_spec, _row_spec,
              _w_spec, _b_spec],
    out_specs=_row_spec)
_tc3 = pl.pallas_call(
    _tc3_body, grid=_GRID, out_shape=_OUT,
    in_specs=[_row_spec, _row_spec, _row_spec, _row_spec, _row_spec, _b_spec],
    out_specs=_row_spec)


def kernel(x, edge_index, W1, b1, W2, b2):
    ei = edge_index.astype(jnp.int32)
    src = ei[0].reshape(NW * NCH, CN, KB)
    dst = ei[1].reshape(NW * NCH, CN, KB)
    x_p = jnp.pad(x, ((0, NP - N), (0, 0)))
    onesD = jnp.zeros((KB, D), jnp.float32).at[:, 0].set(1.0)
    zD = jnp.zeros((KB, D), jnp.float32)
    b1r = b1.reshape(1, D)
    b2r = b2.reshape(1, D)

    degD = _deg_kernel(dst, onesD, zD)
    dega, degb = degD[:NP], degD[NP:]

    g1 = _tc1(x_p, W1, dega, degb)
    acc1 = _agg_kernel(g1, src, dst, zD)
    g2 = _tc2(acc1[:NP], acc1[NP:], g1, dega, degb, W2, b1r)
    acc2 = _agg_kernel(g2, src, dst, zD)
    out = _tc3(acc2[:NP], acc2[NP:], g2, dega, degb, b2r)
    return out[:N]


# speedup vs baseline: 23.0920x; 1.0402x over previous
import functools

import jax
import jax.numpy as jnp
from jax import lax
from jax.experimental import pallas as pl
from jax.experimental.pallas import tpu as pltpu
from jax.experimental.pallas import tpu_sc as plsc

N = 10000
NP = 10240
E = 320000
D = 128
NC, NS = 2, 16
NW = NC * NS
EPW = E // NW
KB = 100
NJ = EPW // KB
CN = 25
NCH = NJ // CN
RPT = NP // NS
WB = 80
BM = 512

_mesh = plsc.VectorSubcoreMesh(
    core_axis_name="c", subcore_axis_name="s", num_cores=NC, num_subcores=NS)


@functools.partial(
    pl.kernel,
    out_type=jax.ShapeDtypeStruct((NC * NP, D), jnp.float32),
    mesh=_mesh,
    scratch_types=[
        pltpu.VMEM((CN, KB), jnp.int32),
        pltpu.VMEM((KB, D), jnp.float32),
        pltpu.VMEM((WB, D), jnp.float32),
        pltpu.VMEM_SHARED((NP, D), jnp.float32),
    ],
)
def _deg_kernel(dst_hbm, ones_hbm, zD_hbm, out_hbm, dst_v, ones_v, z_v, acc_sh):
    cid = lax.axis_index("c")
    sid = lax.axis_index("s")
    wid = cid * NS + sid
    pltpu.sync_copy(ones_hbm, ones_v)
    pltpu.sync_copy(zD_hbm, z_v)
    rows0 = sid * RPT
    for i in range(RPT // WB):
        pltpu.sync_copy(z_v, acc_sh.at[pl.ds(rows0 + i * WB, WB)])
    plsc.subcore_barrier()

    def chunk_body(c, carry):
        pltpu.sync_copy(dst_hbm.at[wid * NCH + c], dst_v)

        def body(j, cc):
            pltpu.sync_copy(ones_v, acc_sh.at[dst_v.at[j]], add=True)
            return cc

        lax.fori_loop(0, CN, body, 0)
        return carry

    lax.fori_loop(0, NCH, chunk_body, 0)
    plsc.subcore_barrier()
    for i in range(RPT // WB):
        pltpu.sync_copy(acc_sh.at[pl.ds(rows0 + i * WB, WB)], z_v)
        pltpu.sync_copy(
            z_v, out_hbm.at[pl.ds(cid * NP + rows0 + i * WB, WB)])


@functools.partial(
    pl.kernel,
    out_type=jax.ShapeDtypeStruct((NC * NP, D), jnp.float32),
    mesh=_mesh,
    scratch_types=[
        pltpu.VMEM((CN, KB), jnp.int32),
        pltpu.VMEM((CN, KB), jnp.int32),
        pltpu.VMEM((KB, D), jnp.float32),
        pltpu.VMEM((KB, D), jnp.float32),
        pltpu.VMEM_SHARED((NP, D), jnp.float32),
        pltpu.SemaphoreType.DMA,
        pltpu.SemaphoreType.DMA,
    ],
)
def _agg_kernel(g_hbm, src_hbm, dst_hbm, zD_hbm, out_hbm,
                src_v, dst_v, rows_a, rows_b, acc_sh, sem_a, sem_b):
    cid = lax.axis_index("c")
    sid = lax.axis_index("s")
    wid = cid * NS + sid
    rows0 = sid * RPT
    pltpu.sync_copy(zD_hbm, rows_a.at[pl.ds(0, WB)])
    for i in range(RPT // WB):
        pltpu.sync_copy(rows_a.at[pl.ds(0, WB)],
                        acc_sh.at[pl.ds(rows0 + i * WB, WB)])
    plsc.subcore_barrier()

    def chunk_body(c, carry):
        pltpu.sync_copy(src_hbm.at[wid * NCH + c], src_v)
        pltpu.sync_copy(dst_hbm.at[wid * NCH + c], dst_v)
        pltpu.async_copy(g_hbm.at[src_v.at[0]], rows_a, sem_a)

        def pair(j2, cc):
            ja = 2 * j2
            jb = ja + 1
            pltpu.async_copy(g_hbm.at[src_v.at[jb]], rows_b, sem_b)
            pltpu.make_async_copy(g_hbm.at[src_v.at[ja]], rows_a, sem_a).wait()
            pltpu.sync_copy(rows_a, acc_sh.at[dst_v.at[ja]], add=True)
            pltpu.async_copy(g_hbm.at[src_v.at[ja + 2]], rows_a, sem_a)
            pltpu.make_async_copy(g_hbm.at[src_v.at[jb]], rows_b, sem_b).wait()
            pltpu.sync_copy(rows_b, acc_sh.at[dst_v.at[jb]], add=True)
            return cc

        lax.fori_loop(0, CN // 2, pair, 0)
        pltpu.make_async_copy(g_hbm.at[src_v.at[CN - 1]], rows_a, sem_a).wait()
        pltpu.sync_copy(rows_a, acc_sh.at[dst_v.at[CN - 1]], add=True)
        return carry

    lax.fori_loop(0, NCH, chunk_body, 0)

    plsc.subcore_barrier()
    for i in range(RPT // WB):
        pltpu.sync_copy(acc_sh.at[pl.ds(rows0 + i * WB, WB)],
                        rows_a.at[pl.ds(0, WB)])
        pltpu.sync_copy(
            rows_a.at[pl.ds(0, WB)],
            out_hbm.at[pl.ds(cid * NP + rows0 + i * WB, WB)])


def _dinv_block(dega_ref, degb_ref, i):
    deg = dega_ref[:, :1] + degb_ref[:, :1]
    rowid = i * BM + lax.broadcasted_iota(jnp.int32, (BM, 1), 0)
    deg = deg + jnp.where(rowid < N, 1.0, 0.0)
    return jnp.where(deg > 0, lax.rsqrt(deg), 0.0)


def _tc1_body(x_ref, w_ref, dega_ref, degb_ref, g_ref):
    dinv = _dinv_block(dega_ref, degb_ref, pl.program_id(0))
    h = jnp.dot(x_ref[...], w_ref[...], preferred_element_type=jnp.float32)
    g_ref[...] = h * dinv


def _tc2_body(acca_ref, accb_ref, g1_ref, dega_ref, degb_ref, w_ref, b_ref,
              g2_ref):
    dinv = _dinv_block(dega_ref, degb_ref, pl.program_id(0))
    z = dinv * (acca_ref[...] + accb_ref[...] + g1_ref[...]) + b_ref[...]
    z = jnp.maximum(z, 0.0)
    h2 = jnp.dot(z, w_ref[...], preferred_element_type=jnp.float32)
    g2_ref[...] = h2 * dinv


def _tc3_body(acca_ref, accb_ref, g2_ref, dega_ref, degb_ref, b_ref, out_ref):
    dinv = _dinv_block(dega_ref, degb_ref, pl.program_id(0))
    out_ref[...] = dinv * (acca_ref[...] + accb_ref[...] + g2_ref[...]) \
        + b_ref[...]


_row_spec = pl.BlockSpec((BM, D), lambda i: (i, 0))
_w_spec = pl.BlockSpec((D, D), lambda i: (0, 0))
_b_spec = pl.BlockSpec((1, D), lambda i: (0, 0))
_GRID = (NP // BM,)
_OUT = jax.ShapeDtypeStruct((NP, D), jnp.float32)

_tc1 = pl.pallas_call(
    _tc1_body, grid=_GRID, out_shape=_OUT,
    in_specs=[_row_spec, _w_spec, _row_spec, _row_spec],
    out_specs=_row_spec)
_tc2 = pl.pallas_call(
    _tc2_body, grid=_GRID, out_shape=_OUT,
    in_specs=[_row_spec, _row_spec, _row_spec, _row_spec, _row_spec,
              _w_spec, _b_spec],
    out_specs=_row_spec)
_tc3 = pl.pallas_call(
    _tc3_body, grid=_GRID, out_shape=_OUT,
    in_specs=[_row_spec, _row_spec, _row_spec, _row_spec, _row_spec, _b_spec],
    out_specs=_row_spec)


def kernel(x, edge_index, W1, b1, W2, b2):
    ei = edge_index.astype(jnp.int32)
    src = ei[0].reshape(NW * NCH, CN, KB)
    dst = ei[1].reshape(NW * NCH, CN, KB)
    x_p = jnp.pad(x, ((0, NP - N), (0, 0)))
    onesD = jnp.zeros((KB, D), jnp.float32).at[:, 0].set(1.0)
    zD = jnp.zeros((WB, D), jnp.float32)
    b1r = b1.reshape(1, D)
    b2r = b2.reshape(1, D)

    degD = _deg_kernel(dst, onesD, zD)
    dega, degb = degD[:NP], degD[NP:]

    g1 = _tc1(x_p, W1, dega, degb)
    acc1 = _agg_kernel(g1, src, dst, zD)
    g2 = _tc2(acc1[:NP], acc1[NP:], g1, dega, degb, W2, b1r)
    acc2 = _agg_kernel(g2, src, dst, zD)
    out = _tc3(acc2[:NP], acc2[NP:], g2, dega, degb, b2r)
    return out[:N]


# speedup vs baseline: 23.2970x; 1.0089x over previous
import functools

import jax
import jax.numpy as jnp
from jax import lax
from jax.experimental import pallas as pl
from jax.experimental.pallas import tpu as pltpu
from jax.experimental.pallas import tpu_sc as plsc

N = 10000
NP = 10240
E = 320000
D = 128
NC, NS = 2, 16
NW = NC * NS
EPW = E // NW
KB = 100
NJ = EPW // KB
CN = 25
NCH = NJ // CN
KBA = 50
NJA = EPW // KBA
CNA = 40
NCHA = NJA // CNA
RPT = NP // NS
WB = 80
BM = 512

_mesh = plsc.VectorSubcoreMesh(
    core_axis_name="c", subcore_axis_name="s", num_cores=NC, num_subcores=NS)


@functools.partial(
    pl.kernel,
    out_type=jax.ShapeDtypeStruct((NC * NP, D), jnp.float32),
    mesh=_mesh,
    scratch_types=[
        pltpu.VMEM((CN, KB), jnp.int32),
        pltpu.VMEM((KB, D), jnp.float32),
        pltpu.VMEM((WB, D), jnp.float32),
        pltpu.VMEM_SHARED((NP, D), jnp.float32),
    ],
)
def _deg_kernel(dst_hbm, ones_hbm, zD_hbm, out_hbm, dst_v, ones_v, z_v, acc_sh):
    cid = lax.axis_index("c")
    sid = lax.axis_index("s")
    wid = cid * NS + sid
    pltpu.sync_copy(ones_hbm, ones_v)
    pltpu.sync_copy(zD_hbm, z_v)
    rows0 = sid * RPT
    for i in range(RPT // WB):
        pltpu.sync_copy(z_v, acc_sh.at[pl.ds(rows0 + i * WB, WB)])
    plsc.subcore_barrier()

    def chunk_body(c, carry):
        pltpu.sync_copy(dst_hbm.at[wid * NCH + c], dst_v)

        def body(j, cc):
            pltpu.sync_copy(ones_v, acc_sh.at[dst_v.at[j]], add=True)
            return cc

        lax.fori_loop(0, CN, body, 0)
        return carry

    lax.fori_loop(0, NCH, chunk_body, 0)
    plsc.subcore_barrier()
    for i in range(RPT // WB):
        pltpu.sync_copy(acc_sh.at[pl.ds(rows0 + i * WB, WB)], z_v)
        pltpu.sync_copy(
            z_v, out_hbm.at[pl.ds(cid * NP + rows0 + i * WB, WB)])


@functools.partial(
    pl.kernel,
    out_type=jax.ShapeDtypeStruct((NC * NP, D), jnp.float32),
    mesh=_mesh,
    scratch_types=[
        pltpu.VMEM((CNA, KBA), jnp.int32),
        pltpu.VMEM((CNA, KBA), jnp.int32),
        pltpu.VMEM((WB, D), jnp.float32),
        [pltpu.VMEM((KBA, D), jnp.float32) for _ in range(4)],
        pltpu.VMEM_SHARED((NP, D), jnp.float32),
        [pltpu.SemaphoreType.DMA for _ in range(4)],
    ],
)
def _agg_kernel(g_hbm, src_hbm, dst_hbm, zD_hbm, out_hbm,
                src_v, dst_v, stage_v, bufs, acc_sh, sems):
    cid = lax.axis_index("c")
    sid = lax.axis_index("s")
    wid = cid * NS + sid
    rows0 = sid * RPT
    pltpu.sync_copy(zD_hbm, stage_v)
    for i in range(RPT // WB):
        pltpu.sync_copy(stage_v, acc_sh.at[pl.ds(rows0 + i * WB, WB)])
    plsc.subcore_barrier()

    def chunk_body(c, carry):
        pltpu.sync_copy(src_hbm.at[wid * NCHA + c], src_v)
        pltpu.sync_copy(dst_hbm.at[wid * NCHA + c], dst_v)
        for t in range(3):
            pltpu.async_copy(g_hbm.at[src_v.at[t]], bufs[t], sems[t])

        def quad(i, cc):
            s = 4 * i
            for t in range(4):
                j = s + t
                pltpu.make_async_copy(
                    g_hbm.at[src_v.at[j]], bufs[t], sems[t]).wait()
                pltpu.sync_copy(bufs[t], acc_sh.at[dst_v.at[j]], add=True)

                @pl.when(j + 3 < CNA)
                def _():
                    pltpu.async_copy(g_hbm.at[src_v.at[j + 3]],
                                     bufs[(t + 3) % 4], sems[(t + 3) % 4])
            return cc

        lax.fori_loop(0, CNA // 4, quad, 0)
        return carry

    lax.fori_loop(0, NCHA, chunk_body, 0)

    plsc.subcore_barrier()
    for i in range(RPT // WB):
        pltpu.sync_copy(acc_sh.at[pl.ds(rows0 + i * WB, WB)], stage_v)
        pltpu.sync_copy(
            stage_v, out_hbm.at[pl.ds(cid * NP + rows0 + i * WB, WB)])


def _dinv_block(dega_ref, degb_ref, i):
    deg = dega_ref[:, :1] + degb_ref[:, :1]
    rowid = i * BM + lax.broadcasted_iota(jnp.int32, (BM, 1), 0)
    deg = deg + jnp.where(rowid < N, 1.0, 0.0)
    return jnp.where(deg > 0, lax.rsqrt(deg), 0.0)


def _tc1_body(x_ref, w_ref, dega_ref, degb_ref, g_ref):
    dinv = _dinv_block(dega_ref, degb_ref, pl.program_id(0))
    h = jnp.dot(x_ref[...], w_ref[...], preferred_element_type=jnp.float32)
    g_ref[...] = h * dinv


def _tc2_body(acca_ref, accb_ref, g1_ref, dega_ref, degb_ref, w_ref, b_ref,
              g2_ref):
    dinv = _dinv_block(dega_ref, degb_ref, pl.program_id(0))
    z = dinv * (acca_ref[...] + accb_ref[...] + g1_ref[...]) + b_ref[...]
    z = jnp.maximum(z, 0.0)
    h2 = jnp.dot(z, w_ref[...], preferred_element_type=jnp.float32)
    g2_ref[...] = h2 * dinv


def _tc3_body(acca_ref, accb_ref, g2_ref, dega_ref, degb_ref, b_ref, out_ref):
    dinv = _dinv_block(dega_ref, degb_ref, pl.program_id(0))
    out_ref[...] = dinv * (acca_ref[...] + accb_ref[...] + g2_ref[...]) \
        + b_ref[...]


_row_spec = pl.BlockSpec((BM, D), lambda i: (i, 0))
_w_spec = pl.BlockSpec((D, D), lambda i: (0, 0))
_b_spec = pl.BlockSpec((1, D), lambda i: (0, 0))
_GRID = (NP // BM,)
_OUT = jax.ShapeDtypeStruct((NP, D), jnp.float32)

_tc1 = pl.pallas_call(
    _tc1_body, grid=_GRID, out_shape=_OUT,
    in_specs=[_row_spec, _w_spec, _row_spec, _row_spec],
    out_specs=_row_spec)
_tc2 = pl.pallas_call(
    _tc2_body, grid=_GRID, out_shape=_OUT,
    in_specs=[_row_spec, _row_spec, _row_spec, _row_spec, _row_spec,
              _w_spec, _b_spec],
    out_specs=_row_spec)
_tc3 = pl.pallas_call(
    _tc3_body, grid=_GRID, out_shape=_OUT,
    in_specs=[_row_spec, _row_spec, _row_spec, _row_spec, _row_spec, _b_spec],
    out_specs=_row_spec)


def kernel(x, edge_index, W1, b1, W2, b2):
    ei = edge_index.astype(jnp.int32)
    src = ei[0].reshape(NW * NCHA, CNA, KBA)
    dst = ei[1].reshape(NW * NCHA, CNA, KBA)
    dst_deg = ei[1].reshape(NW * NCH, CN, KB)
    x_p = jnp.pad(x, ((0, NP - N), (0, 0)))
    onesD = jnp.zeros((KB, D), jnp.float32).at[:, 0].set(1.0)
    zD = jnp.zeros((WB, D), jnp.float32)
    b1r = b1.reshape(1, D)
    b2r = b2.reshape(1, D)

    degD = _deg_kernel(dst_deg, onesD, zD)
    dega, degb = degD[:NP], degD[NP:]

    g1 = _tc1(x_p, W1, dega, degb)
    acc1 = _agg_kernel(g1, src, dst, zD)
    g2 = _tc2(acc1[:NP], acc1[NP:], g1, dega, degb, W2, b1r)
    acc2 = _agg_kernel(g2, src, dst, zD)
    out = _tc3(acc2[:NP], acc2[NP:], g2, dega, degb, b2r)
    return out[:N]
